# baseline (speedup 1.0000x reference)
import jax
import jax.numpy as jnp
from jax.experimental import pallas as pl
from jax.experimental.pallas import tpu as pltpu

D3 = 256
D_MODEL = 768
SEQ = 81
REP = 32


def _pe_build_kernel(rows_ref, cols_ref, boxes_ref,
                     row_tab_ref, col_tab_ref, box_tab_ref, pe_ref):
    iota = jax.lax.broadcasted_iota(jnp.int32, (SEQ, 9), 1)
    oh_rows = (rows_ref[...] == iota).astype(jnp.float32)
    oh_cols = (cols_ref[...] == iota).astype(jnp.float32)
    oh_boxes = (boxes_ref[...] == iota).astype(jnp.float32)
    pe_ref[:, 0:D3] = jnp.dot(oh_rows, row_tab_ref[...],
                              preferred_element_type=jnp.float32)
    pe_ref[:, D3:2 * D3] = jnp.dot(oh_cols, col_tab_ref[...],
                                   preferred_element_type=jnp.float32)
    pe_ref[:, 2 * D3:D_MODEL] = jnp.dot(oh_boxes, box_tab_ref[...],
                                        preferred_element_type=jnp.float32)


def _add_kernel(pe_ref, x_ref, out_ref):
    out_ref[...] = x_ref[...] + pe_ref[...][None, :, :]


@jax.jit
def kernel(x, row_table, col_table, box_table, rows, cols, boxes):
    pe = pl.pallas_call(
        _pe_build_kernel,
        out_shape=jax.ShapeDtypeStruct((SEQ, D_MODEL), jnp.float32),
    )(rows.reshape(SEQ, 1), cols.reshape(SEQ, 1), boxes.reshape(SEQ, 1),
      row_table, col_table, box_table)

    b = x.shape[0]
    return pl.pallas_call(
        _add_kernel,
        grid=(b // REP,),
        in_specs=[
            pl.BlockSpec((SEQ, D_MODEL), lambda i: (0, 0)),
            pl.BlockSpec((REP, SEQ, D_MODEL), lambda i: (i, 0, 0)),
        ],
        out_specs=pl.BlockSpec((REP, SEQ, D_MODEL), lambda i: (i, 0, 0)),
        out_shape=jax.ShapeDtypeStruct(x.shape, x.dtype),
        compiler_params=pltpu.CompilerParams(
            dimension_semantics=("arbitrary",),
        ),
    )(pe, x)
